# baseline (speedup 1.0000x reference)
import functools
import jax
import jax.numpy as jnp
from jax import lax
from jax.experimental import pallas as pl
from jax.experimental.pallas import tpu as pltpu
from jax.experimental.pallas import tpu_sc as plsc

AFL = 128
NBR = 16
DF = 2 * AFL + NBR
DP = 288
EPS = 1e-5


def _pack_pair(lo_cols, hi_cols):
    lo = lax.shift_right_logical(lax.bitcast_convert_type(lo_cols, jnp.int32), 16)
    hi = jnp.bitwise_and(lax.bitcast_convert_type(hi_cols, jnp.int32),
                         jnp.int32(-65536))
    return jnp.bitwise_or(lo, hi)


def _qproj_body(atom_ref, w1_ref, qta_ref, qtb_ref, *, bn):
    q = jnp.dot(atom_ref[:], w1_ref[:], preferred_element_type=jnp.float32)
    qr = q.astype(jnp.bfloat16).astype(jnp.float32)
    qta_ref[:] = _pack_pair(qr[:, :AFL], qr[:, AFL:2 * AFL])
    qtb_ref[:] = _pack_pair(qr[:, 2 * AFL:2 * AFL + 8], qr[:, 2 * AFL + 8:])


def _sc_gather(table_a, table_b_flat, idx_flat, idx_flat_b):
    E = idx_flat.shape[0]
    NW = 32
    per_w = E // NW
    CH = 80
    n_ch = per_w // CH
    nb_words = table_b_flat.shape[0]
    mesh = plsc.VectorSubcoreMesh(core_axis_name="c", subcore_axis_name="s")

    @functools.partial(
        pl.kernel,
        mesh=mesh,
        compiler_params=pltpu.CompilerParams(needs_layout_passes=False),
        out_type=[
            jax.ShapeDtypeStruct((E, 128), jnp.int32),
            jax.ShapeDtypeStruct((E, 8), jnp.int32),
        ],
        scratch_types=[
            pltpu.VMEM((per_w,), jnp.int32),
            pltpu.VMEM((per_w,), jnp.int32),
            pltpu.VMEM((CH, 128), jnp.int32),
            pltpu.VMEM((CH, 8), jnp.int32),
            pltpu.VMEM((nb_words,), jnp.int32),
            pltpu.SemaphoreType.DMA,
        ],
    )
    def gather_kernel(ta_hbm, tb_hbm, idxm_hbm, idxn_hbm, outa_hbm, outb_hbm,
                      idxm_v, idxn_v, rows_a, rows_b, tb_v, sem):
        wid = lax.axis_index("s") * 2 + lax.axis_index("c")
        base = pl.multiple_of(wid * per_w, 8)
        pltpu.sync_copy(tb_hbm, tb_v)
        pltpu.sync_copy(idxm_hbm.at[pl.ds(base, per_w)], idxm_v)
        pltpu.sync_copy(idxn_hbm.at[pl.ds(base, per_w)], idxn_v)
        lane = lax.iota(jnp.int32, 16)

        def body(i, carry):
            off = pl.multiple_of(i * CH, 8)
            ca = pltpu.async_copy(ta_hbm.at[idxm_v.at[pl.ds(off, CH)]], rows_a, sem)
            for e16 in range(CH // 16):
                jv = idxn_v[pl.ds(off + e16 * 16, 16)] * 8
                row = lane + e16 * 16
                for c in range(8):
                    v = plsc.load_gather(tb_v, [jv + c])
                    plsc.store_scatter(rows_b, [row, jnp.full((16,), c, jnp.int32)], v)
            ca.wait()
            pltpu.sync_copy(rows_a, outa_hbm.at[pl.ds(base + off, CH)])
            pltpu.sync_copy(rows_b, outb_hbm.at[pl.ds(base + off, CH)])
            return carry

        lax.fori_loop(0, n_ch, body, 0)

    return gather_kernel(table_a, table_b_flat, idx_flat, idx_flat_b)


def _unpack_pair(words):
    lo_f = lax.bitcast_convert_type(lax.shift_left(words, 16), jnp.float32)
    hi_f = lax.bitcast_convert_type(
        jnp.bitwise_and(words, jnp.int32(-65536)), jnp.float32)
    return lo_f, hi_f


def _compute_parts(atom_ref, aga_ref, agb_ref, nbr_ref, w0_ref, w2_ref, b_ref,
                   bn, m):
    sp = jnp.dot(atom_ref[:], w0_ref[:], preferred_element_type=jnp.float32)
    sp = sp + b_ref[:]
    nbr_nm = nbr_ref[:]
    nbr_mm = jnp.transpose(nbr_nm, (1, 0, 2)).reshape(m * bn, NBR)
    e_fc = jnp.dot(nbr_mm, w2_ref[:, :2 * AFL],
                   preferred_element_type=jnp.float32)
    e_n = jnp.dot(nbr_nm.reshape(bn * m, NBR), w2_ref[:, 2 * AFL:],
                  preferred_element_type=jnp.float32)
    qf, qc = _unpack_pair(aga_ref[:].reshape(m * bn, AFL))
    qn_lo, qn_hi = _unpack_pair(agb_ref[:].reshape(bn * m, 8))

    qn = jnp.concatenate([qn_lo, qn_hi], axis=1)

    f = (jnp.broadcast_to(sp[None, :, :AFL], (m, bn, AFL)).reshape(m * bn, AFL)
         + qf + e_fc[:, :AFL])
    c = (jnp.broadcast_to(sp[None, :, AFL:2 * AFL],
                          (m, bn, AFL)).reshape(m * bn, AFL)
         + qc + e_fc[:, AFL:])
    nw = (jnp.broadcast_to(sp[:, None, 2 * AFL:],
                           (bn, m, NBR)).reshape(bn * m, NBR)
          + qn + e_n)
    return f, c, nw, nbr_nm


def _mm_stats_body(atom_ref, aga_ref, agb_ref, nbr_ref, w0_ref, w2_ref, b_ref,
                   s_ref, q_ref, *, bn, m):
    f, c, nw, _ = _compute_parts(atom_ref, aga_ref, agb_ref, nbr_ref,
                                 w0_ref, w2_ref, b_ref, bn, m)

    @pl.when(pl.program_id(0) == 0)
    def _():
        s_ref[:] = jnp.zeros_like(s_ref)
        q_ref[:] = jnp.zeros_like(q_ref)

    s_ref[:] += jnp.concatenate(
        [jnp.sum(f, axis=0, keepdims=True),
         jnp.sum(c, axis=0, keepdims=True),
         jnp.sum(nw, axis=0, keepdims=True)], axis=1)
    q_ref[:] += jnp.concatenate(
        [jnp.sum(f * f, axis=0, keepdims=True),
         jnp.sum(c * c, axis=0, keepdims=True),
         jnp.sum(nw * nw, axis=0, keepdims=True)], axis=1)


def _apply_body(atom_ref, aga_ref, agb_ref, nbr_ref, w0_ref, w2_ref, b_ref,
                s_ref, q_ref, g1_ref, b1_ref,
                ns_ref, nn_ref, s2_ref, q2_ref, *, bn, m, nm_total):
    f, c, nw, nbr_nm = _compute_parts(atom_ref, aga_ref, agb_ref, nbr_ref,
                                      w0_ref, w2_ref, b_ref, bn, m)

    mean = s_ref[:] / nm_total
    var = q_ref[:] / nm_total - mean * mean
    a1 = g1_ref[:] * lax.rsqrt(var + EPS)
    c1 = b1_ref[:] - mean * a1

    fb = (f * a1[:, :AFL] + c1[:, :AFL]).reshape(m, bn, AFL)
    cb = (c * a1[:, AFL:2 * AFL] + c1[:, AFL:2 * AFL]).reshape(m, bn, AFL)
    nwb = (nw * a1[:, 2 * AFL:] + c1[:, 2 * AFL:]).reshape(bn, m, NBR)

    mx = fb[0]
    for j in range(1, m):
        mx = jnp.maximum(mx, fb[j])
    z = jnp.zeros((bn, AFL), jnp.float32)
    acc = jnp.zeros((bn, AFL), jnp.float32)
    for j in range(m):
        e = jnp.exp(fb[j] - mx)
        z = z + e
        acc = acc + e * jnp.maximum(cb[j], 0.0)
    ns = acc / z
    ns_ref[:] = ns
    nn_ref[:] = nwb + nbr_nm

    @pl.when(pl.program_id(0) == 0)
    def _():
        s2_ref[:] = jnp.zeros_like(s2_ref)
        q2_ref[:] = jnp.zeros_like(q2_ref)

    s2_ref[:] += jnp.sum(ns, axis=0, keepdims=True)
    q2_ref[:] += jnp.sum(ns * ns, axis=0, keepdims=True)


def _bn2_body(atom_ref, ns_ref, s2_ref, q2_ref, g2_ref, b2_ref, out_ref, *, n_total):
    mean = s2_ref[:] / n_total
    var = q2_ref[:] / n_total - mean * mean
    a2 = g2_ref[:] * lax.rsqrt(var + EPS)
    c2 = b2_ref[:] - mean * a2
    out_ref[:] = atom_ref[:] + ns_ref[:] * a2 + c2


def kernel(atom_in_fea, nbr_fea, nbr_fea_idx, W, b, g1, b1, g2, b2):
    N, M = nbr_fea_idx.shape
    E = N * M

    idx32 = nbr_fea_idx.astype(jnp.int32)
    idx_m = idx32.T.reshape(E)
    idx_n = idx32.reshape(E)
    w0 = W[:AFL, :]
    w1 = W[AFL:2 * AFL, :]
    w2 = W[2 * AFL:, :]
    b2d = b.reshape(1, DF)

    BQ = 1000
    qta, qtb = pl.pallas_call(
        functools.partial(_qproj_body, bn=BQ),
        grid=(N // BQ,),
        in_specs=[
            pl.BlockSpec((BQ, AFL), lambda i: (i, 0)),
            pl.BlockSpec((AFL, DF), lambda i: (0, 0)),
        ],
        out_specs=[
            pl.BlockSpec((BQ, AFL), lambda i: (i, 0)),
            pl.BlockSpec((BQ, 8), lambda i: (i, 0)),
        ],
        out_shape=[
            jax.ShapeDtypeStruct((N, AFL), jnp.int32),
            jax.ShapeDtypeStruct((N, 8), jnp.int32),
        ],
    )(atom_in_fea, w1)

    aga, agb = _sc_gather(qta, qtb.reshape(N * 8), idx_m, idx_n)
    aga3 = aga.reshape(M, N, AFL)
    agb3 = agb.reshape(N, M, 8)

    BN = 200
    grid_a = N // BN

    dense_specs = [
        pl.BlockSpec((BN, AFL), lambda i: (i, 0)),
        pl.BlockSpec((M, BN, AFL), lambda i: (0, i, 0)),
        pl.BlockSpec((BN, M, 8), lambda i: (i, 0, 0)),
        pl.BlockSpec((BN, M, NBR), lambda i: (i, 0, 0)),
        pl.BlockSpec((AFL, DF), lambda i: (0, 0)),
        pl.BlockSpec((NBR, DF), lambda i: (0, 0)),
        pl.BlockSpec((1, DF), lambda i: (0, 0)),
    ]
    s, q = pl.pallas_call(
        functools.partial(_mm_stats_body, bn=BN, m=M),
        grid=(grid_a,),
        in_specs=dense_specs,
        out_specs=[
            pl.BlockSpec((1, DF), lambda i: (0, 0)),
            pl.BlockSpec((1, DF), lambda i: (0, 0)),
        ],
        out_shape=[
            jax.ShapeDtypeStruct((1, DF), jnp.float32),
            jax.ShapeDtypeStruct((1, DF), jnp.float32),
        ],
    )(atom_in_fea, aga3, agb3, nbr_fea, w0, w2, b2d)

    ns, nn, s2, q2 = pl.pallas_call(
        functools.partial(_apply_body, bn=BN, m=M, nm_total=float(E)),
        grid=(grid_a,),
        in_specs=dense_specs + [
            pl.BlockSpec((1, DF), lambda i: (0, 0)),
            pl.BlockSpec((1, DF), lambda i: (0, 0)),
            pl.BlockSpec((1, DF), lambda i: (0, 0)),
            pl.BlockSpec((1, DF), lambda i: (0, 0)),
        ],
        out_specs=[
            pl.BlockSpec((BN, AFL), lambda i: (i, 0)),
            pl.BlockSpec((BN, M, NBR), lambda i: (i, 0, 0)),
            pl.BlockSpec((1, AFL), lambda i: (0, 0)),
            pl.BlockSpec((1, AFL), lambda i: (0, 0)),
        ],
        out_shape=[
            jax.ShapeDtypeStruct((N, AFL), jnp.float32),
            jax.ShapeDtypeStruct((N, M, NBR), jnp.float32),
            jax.ShapeDtypeStruct((1, AFL), jnp.float32),
            jax.ShapeDtypeStruct((1, AFL), jnp.float32),
        ],
    )(atom_in_fea, aga3, agb3, nbr_fea, w0, w2, b2d,
      s, q, g1.reshape(1, DF), b1.reshape(1, DF))

    BC = 1000
    out = pl.pallas_call(
        functools.partial(_bn2_body, n_total=float(N)),
        grid=(N // BC,),
        in_specs=[
            pl.BlockSpec((BC, AFL), lambda i: (i, 0)),
            pl.BlockSpec((BC, AFL), lambda i: (i, 0)),
            pl.BlockSpec((1, AFL), lambda i: (0, 0)),
            pl.BlockSpec((1, AFL), lambda i: (0, 0)),
            pl.BlockSpec((1, AFL), lambda i: (0, 0)),
            pl.BlockSpec((1, AFL), lambda i: (0, 0)),
        ],
        out_specs=pl.BlockSpec((BC, AFL), lambda i: (i, 0)),
        out_shape=jax.ShapeDtypeStruct((N, AFL), jnp.float32),
    )(atom_in_fea, ns, s2, q2, g2.reshape(1, AFL), b2.reshape(1, AFL))

    return (out, nn)


# speedup vs baseline: 1.0174x; 1.0174x over previous
import functools
import jax
import jax.numpy as jnp
from jax import lax
from jax.experimental import pallas as pl
from jax.experimental.pallas import tpu as pltpu
from jax.experimental.pallas import tpu_sc as plsc

AFL = 128
NBR = 16
DF = 2 * AFL + NBR
DP = 288
EPS = 1e-5


def _pack_pair(lo_cols, hi_cols):
    lo = lax.shift_right_logical(lax.bitcast_convert_type(lo_cols, jnp.int32), 16)
    hi = jnp.bitwise_and(lax.bitcast_convert_type(hi_cols, jnp.int32),
                         jnp.int32(-65536))
    return jnp.bitwise_or(lo, hi)


def _qproj_body(atom_ref, w1_ref, qta_ref, qtb_ref, *, bn):
    q = jnp.dot(atom_ref[:], w1_ref[:], preferred_element_type=jnp.float32)
    qr = q.astype(jnp.bfloat16).astype(jnp.float32)
    qta_ref[:] = _pack_pair(qr[:, :AFL], qr[:, AFL:2 * AFL])
    qtb_ref[:] = _pack_pair(qr[:, 2 * AFL:2 * AFL + 8], qr[:, 2 * AFL + 8:])


def _sc_gather(table_a, table_b_flat, idx_flat, idx_flat_b):
    E = idx_flat.shape[0]
    NW = 32
    per_w = E // NW
    CH = 80
    n_ch = per_w // CH
    nb_words = table_b_flat.shape[0]
    mesh = plsc.VectorSubcoreMesh(core_axis_name="c", subcore_axis_name="s")

    @functools.partial(
        pl.kernel,
        mesh=mesh,
        compiler_params=pltpu.CompilerParams(needs_layout_passes=False),
        out_type=[
            jax.ShapeDtypeStruct((E, 128), jnp.int32),
            jax.ShapeDtypeStruct((E, 8), jnp.int32),
        ],
        scratch_types=[
            pltpu.VMEM((per_w,), jnp.int32),
            pltpu.VMEM((per_w,), jnp.int32),
            pltpu.VMEM((CH, 128), jnp.int32),
            pltpu.VMEM((CH, 128), jnp.int32),
            pltpu.VMEM((CH, 8), jnp.int32),
            pltpu.VMEM((nb_words,), jnp.int32),
            pltpu.SemaphoreType.DMA,
            pltpu.SemaphoreType.DMA,
        ],
    )
    def gather_kernel(ta_hbm, tb_hbm, idxm_hbm, idxn_hbm, outa_hbm, outb_hbm,
                      idxm_v, idxn_v, rows_a0, rows_a1, rows_b, tb_v,
                      sem0, sem1):
        wid = lax.axis_index("s") * 2 + lax.axis_index("c")
        base = pl.multiple_of(wid * per_w, 8)
        pltpu.sync_copy(tb_hbm, tb_v)
        pltpu.sync_copy(idxm_hbm.at[pl.ds(base, per_w)], idxm_v)
        pltpu.sync_copy(idxn_hbm.at[pl.ds(base, per_w)], idxn_v)
        lane = lax.iota(jnp.int32, 16)
        bufs = (rows_a0, rows_a1)
        sems = (sem0, sem1)

        def start(ch, buf, sem):
            off = pl.multiple_of(ch * CH, 8)
            pltpu.async_copy(ta_hbm.at[idxm_v.at[pl.ds(off, CH)]], buf, sem)

        def finish(ch, buf, sem):
            off = pl.multiple_of(ch * CH, 8)
            for e16 in range(CH // 16):
                jv = idxn_v[pl.ds(off + e16 * 16, 16)] * 8
                row = lane + e16 * 16
                for c in range(8):
                    v = plsc.load_gather(tb_v, [jv + c])
                    plsc.store_scatter(
                        rows_b, [row, jnp.full((16,), c, jnp.int32)], v)
            pltpu.make_async_copy(
                ta_hbm.at[idxm_v.at[pl.ds(off, CH)]], buf, sem).wait()
            pltpu.sync_copy(buf, outa_hbm.at[pl.ds(base + off, CH)])
            pltpu.sync_copy(rows_b, outb_hbm.at[pl.ds(base + off, CH)])

        start(0, bufs[0], sems[0])

        def body(ih, carry):
            for bslot in range(2):
                ch = ih * 2 + bslot
                start(ch + 1, bufs[1 - bslot], sems[1 - bslot])
                finish(ch, bufs[bslot], sems[bslot])
            return carry

        lax.fori_loop(0, n_ch // 2, body, 0)
        finish(n_ch - 1, bufs[0], sems[0])

    return gather_kernel(table_a, table_b_flat, idx_flat, idx_flat_b)


def _unpack_pair(words):
    lo_f = lax.bitcast_convert_type(lax.shift_left(words, 16), jnp.float32)
    hi_f = lax.bitcast_convert_type(
        jnp.bitwise_and(words, jnp.int32(-65536)), jnp.float32)
    return lo_f, hi_f


def _compute_parts(atom_ref, aga_ref, agb_ref, nbr_ref, nbrt_ref,
                   w0_ref, w2_ref, b_ref, bn, m):
    sp = jnp.dot(atom_ref[:], w0_ref[:], preferred_element_type=jnp.float32)
    sp = sp + b_ref[:]
    nbr_nm = nbr_ref[:]
    e_fc = jnp.dot(nbrt_ref[:].reshape(m * bn, NBR), w2_ref[:, :2 * AFL],
                   preferred_element_type=jnp.float32)
    e_n = jnp.dot(nbr_nm.reshape(bn * m, NBR), w2_ref[:, 2 * AFL:],
                  preferred_element_type=jnp.float32)
    qf, qc = _unpack_pair(aga_ref[:].reshape(m * bn, AFL))
    qn_lo, qn_hi = _unpack_pair(agb_ref[:].reshape(bn * m, 8))

    qn = jnp.concatenate([qn_lo, qn_hi], axis=1)

    f = (jnp.broadcast_to(sp[None, :, :AFL], (m, bn, AFL)).reshape(m * bn, AFL)
         + qf + e_fc[:, :AFL])
    c = (jnp.broadcast_to(sp[None, :, AFL:2 * AFL],
                          (m, bn, AFL)).reshape(m * bn, AFL)
         + qc + e_fc[:, AFL:])
    nw = (jnp.broadcast_to(sp[:, None, 2 * AFL:],
                           (bn, m, NBR)).reshape(bn * m, NBR)
          + qn + e_n)
    return f, c, nw, nbr_nm


def _mm_stats_body(atom_ref, aga_ref, agb_ref, nbr_ref, nbrt_ref,
                   w0_ref, w2_ref, b_ref, s_ref, q_ref, *, bn, m):
    f, c, nw, _ = _compute_parts(atom_ref, aga_ref, agb_ref, nbr_ref, nbrt_ref,
                                 w0_ref, w2_ref, b_ref, bn, m)

    @pl.when(pl.program_id(0) == 0)
    def _():
        s_ref[:] = jnp.zeros_like(s_ref)
        q_ref[:] = jnp.zeros_like(q_ref)

    s_ref[:] += jnp.concatenate(
        [jnp.sum(f, axis=0, keepdims=True),
         jnp.sum(c, axis=0, keepdims=True),
         jnp.sum(nw, axis=0, keepdims=True)], axis=1)
    q_ref[:] += jnp.concatenate(
        [jnp.sum(f * f, axis=0, keepdims=True),
         jnp.sum(c * c, axis=0, keepdims=True),
         jnp.sum(nw * nw, axis=0, keepdims=True)], axis=1)


def _apply_body(atom_ref, aga_ref, agb_ref, nbr_ref, nbrt_ref,
                w0_ref, w2_ref, b_ref, s_ref, q_ref, g1_ref, b1_ref,
                ns_ref, nn_ref, s2_ref, q2_ref, *, bn, m, nm_total):
    f, c, nw, nbr_nm = _compute_parts(atom_ref, aga_ref, agb_ref, nbr_ref,
                                      nbrt_ref, w0_ref, w2_ref, b_ref, bn, m)

    mean = s_ref[:] / nm_total
    var = q_ref[:] / nm_total - mean * mean
    a1 = g1_ref[:] * lax.rsqrt(var + EPS)
    c1 = b1_ref[:] - mean * a1

    fb = (f * a1[:, :AFL] + c1[:, :AFL]).reshape(m, bn, AFL)
    cb = (c * a1[:, AFL:2 * AFL] + c1[:, AFL:2 * AFL]).reshape(m, bn, AFL)
    nwb = (nw * a1[:, 2 * AFL:] + c1[:, 2 * AFL:]).reshape(bn, m, NBR)

    mx = fb[0]
    for j in range(1, m):
        mx = jnp.maximum(mx, fb[j])
    z = jnp.zeros((bn, AFL), jnp.float32)
    acc = jnp.zeros((bn, AFL), jnp.float32)
    for j in range(m):
        e = jnp.exp(fb[j] - mx)
        z = z + e
        acc = acc + e * jnp.maximum(cb[j], 0.0)
    ns = acc / z
    ns_ref[:] = ns
    nn_ref[:] = nwb + nbr_nm

    @pl.when(pl.program_id(0) == 0)
    def _():
        s2_ref[:] = jnp.zeros_like(s2_ref)
        q2_ref[:] = jnp.zeros_like(q2_ref)

    s2_ref[:] += jnp.sum(ns, axis=0, keepdims=True)
    q2_ref[:] += jnp.sum(ns * ns, axis=0, keepdims=True)


def _bn2_body(atom_ref, ns_ref, s2_ref, q2_ref, g2_ref, b2_ref, out_ref, *, n_total):
    mean = s2_ref[:] / n_total
    var = q2_ref[:] / n_total - mean * mean
    a2 = g2_ref[:] * lax.rsqrt(var + EPS)
    c2 = b2_ref[:] - mean * a2
    out_ref[:] = atom_ref[:] + ns_ref[:] * a2 + c2


def kernel(atom_in_fea, nbr_fea, nbr_fea_idx, W, b, g1, b1, g2, b2):
    N, M = nbr_fea_idx.shape
    E = N * M

    idx32 = nbr_fea_idx.astype(jnp.int32)
    idx_m = idx32.T.reshape(E)
    idx_n = idx32.reshape(E)
    nbr_t = jnp.transpose(nbr_fea, (1, 0, 2))
    w0 = W[:AFL, :]
    w1 = W[AFL:2 * AFL, :]
    w2 = W[2 * AFL:, :]
    b2d = b.reshape(1, DF)

    BQ = 1000
    qta, qtb = pl.pallas_call(
        functools.partial(_qproj_body, bn=BQ),
        grid=(N // BQ,),
        in_specs=[
            pl.BlockSpec((BQ, AFL), lambda i: (i, 0)),
            pl.BlockSpec((AFL, DF), lambda i: (0, 0)),
        ],
        out_specs=[
            pl.BlockSpec((BQ, AFL), lambda i: (i, 0)),
            pl.BlockSpec((BQ, 8), lambda i: (i, 0)),
        ],
        out_shape=[
            jax.ShapeDtypeStruct((N, AFL), jnp.int32),
            jax.ShapeDtypeStruct((N, 8), jnp.int32),
        ],
    )(atom_in_fea, w1)

    aga, agb = _sc_gather(qta, qtb.reshape(N * 8), idx_m, idx_n)
    aga3 = aga.reshape(M, N, AFL)
    agb3 = agb.reshape(N, M, 8)

    BN = 200
    grid_a = N // BN

    dense_specs = [
        pl.BlockSpec((BN, AFL), lambda i: (i, 0)),
        pl.BlockSpec((M, BN, AFL), lambda i: (0, i, 0)),
        pl.BlockSpec((BN, M, 8), lambda i: (i, 0, 0)),
        pl.BlockSpec((BN, M, NBR), lambda i: (i, 0, 0)),
        pl.BlockSpec((M, BN, NBR), lambda i: (0, i, 0)),
        pl.BlockSpec((AFL, DF), lambda i: (0, 0)),
        pl.BlockSpec((NBR, DF), lambda i: (0, 0)),
        pl.BlockSpec((1, DF), lambda i: (0, 0)),
    ]
    s, q = pl.pallas_call(
        functools.partial(_mm_stats_body, bn=BN, m=M),
        grid=(grid_a,),
        in_specs=dense_specs,
        out_specs=[
            pl.BlockSpec((1, DF), lambda i: (0, 0)),
            pl.BlockSpec((1, DF), lambda i: (0, 0)),
        ],
        out_shape=[
            jax.ShapeDtypeStruct((1, DF), jnp.float32),
            jax.ShapeDtypeStruct((1, DF), jnp.float32),
        ],
    )(atom_in_fea, aga3, agb3, nbr_fea, nbr_t, w0, w2, b2d)

    ns, nn, s2, q2 = pl.pallas_call(
        functools.partial(_apply_body, bn=BN, m=M, nm_total=float(E)),
        grid=(grid_a,),
        in_specs=dense_specs + [
            pl.BlockSpec((1, DF), lambda i: (0, 0)),
            pl.BlockSpec((1, DF), lambda i: (0, 0)),
            pl.BlockSpec((1, DF), lambda i: (0, 0)),
            pl.BlockSpec((1, DF), lambda i: (0, 0)),
        ],
        out_specs=[
            pl.BlockSpec((BN, AFL), lambda i: (i, 0)),
            pl.BlockSpec((BN, M, NBR), lambda i: (i, 0, 0)),
            pl.BlockSpec((1, AFL), lambda i: (0, 0)),
            pl.BlockSpec((1, AFL), lambda i: (0, 0)),
        ],
        out_shape=[
            jax.ShapeDtypeStruct((N, AFL), jnp.float32),
            jax.ShapeDtypeStruct((N, M, NBR), jnp.float32),
            jax.ShapeDtypeStruct((1, AFL), jnp.float32),
            jax.ShapeDtypeStruct((1, AFL), jnp.float32),
        ],
    )(atom_in_fea, aga3, agb3, nbr_fea, nbr_t, w0, w2, b2d,
      s, q, g1.reshape(1, DF), b1.reshape(1, DF))

    BC = 1000
    out = pl.pallas_call(
        functools.partial(_bn2_body, n_total=float(N)),
        grid=(N // BC,),
        in_specs=[
            pl.BlockSpec((BC, AFL), lambda i: (i, 0)),
            pl.BlockSpec((BC, AFL), lambda i: (i, 0)),
            pl.BlockSpec((1, AFL), lambda i: (0, 0)),
            pl.BlockSpec((1, AFL), lambda i: (0, 0)),
            pl.BlockSpec((1, AFL), lambda i: (0, 0)),
            pl.BlockSpec((1, AFL), lambda i: (0, 0)),
        ],
        out_specs=pl.BlockSpec((BC, AFL), lambda i: (i, 0)),
        out_shape=jax.ShapeDtypeStruct((N, AFL), jnp.float32),
    )(atom_in_fea, ns, s2, q2, g2.reshape(1, AFL), b2.reshape(1, AFL))

    return (out, nn)


# speedup vs baseline: 1.2028x; 1.1823x over previous
import functools
import jax
import jax.numpy as jnp
from jax import lax
from jax.experimental import pallas as pl
from jax.experimental.pallas import tpu as pltpu
from jax.experimental.pallas import tpu_sc as plsc

AFL = 128
NBR = 16
DF = 2 * AFL + NBR
EPS = 1e-5


def _pack_pair(lo_cols, hi_cols):
    lo = lax.shift_right_logical(lax.bitcast_convert_type(lo_cols, jnp.int32), 16)
    hi = jnp.bitwise_and(lax.bitcast_convert_type(hi_cols, jnp.int32),
                         jnp.int32(-65536))
    return jnp.bitwise_or(lo, hi)


def _qproj_body(atom_ref, w1_ref, qta_ref, qtb_ref, *, bn):
    q = jnp.dot(atom_ref[:], w1_ref[:], preferred_element_type=jnp.float32)
    qr = q.astype(jnp.bfloat16).astype(jnp.float32)
    qta_ref[:] = _pack_pair(qr[:, :AFL], qr[:, AFL:2 * AFL])
    qtb_ref[:] = _pack_pair(qr[:, 2 * AFL:2 * AFL + 8], qr[:, 2 * AFL + 8:])


def _sc_gather(table_a, table_b_flat, idx_flat):
    E = idx_flat.shape[0]
    NW = 32
    per_w = E // NW
    CH = 80
    n_ch = per_w // CH
    nb_words = table_b_flat.shape[0]
    mesh = plsc.VectorSubcoreMesh(core_axis_name="c", subcore_axis_name="s")

    @functools.partial(
        pl.kernel,
        mesh=mesh,
        compiler_params=pltpu.CompilerParams(needs_layout_passes=False),
        out_type=[
            jax.ShapeDtypeStruct((E, 128), jnp.int32),
            jax.ShapeDtypeStruct((E * 8,), jnp.int32),
        ],
        scratch_types=[
            pltpu.VMEM((per_w,), jnp.int32),
            pltpu.VMEM((CH, 128), jnp.int32),
            pltpu.VMEM((CH, 128), jnp.int32),
            pltpu.VMEM((CH * 8,), jnp.int32),
            pltpu.VMEM((nb_words,), jnp.int32),
            pltpu.SemaphoreType.DMA,
            pltpu.SemaphoreType.DMA,
        ],
    )
    def gather_kernel(ta_hbm, tb_hbm, idx_hbm, outa_hbm, outb_hbm,
                      idx_v, rows_a0, rows_a1, rows_b, tb_v, sem0, sem1):
        wid = lax.axis_index("s") * 2 + lax.axis_index("c")
        base = pl.multiple_of(wid * per_w, 8)
        pltpu.sync_copy(tb_hbm, tb_v)
        pltpu.sync_copy(idx_hbm.at[pl.ds(base, per_w)], idx_v)
        lane = lax.iota(jnp.int32, 16)
        bufs = (rows_a0, rows_a1)
        sems = (sem0, sem1)

        def start(ch, buf, sem):
            off = pl.multiple_of(ch * CH, 8)
            pltpu.async_copy(ta_hbm.at[idx_v.at[pl.ds(off, CH)]], buf, sem)

        def finish(ch, buf, sem):
            off = pl.multiple_of(ch * CH, 8)
            for e16 in range(CH // 16):
                jv = idx_v[pl.ds(off + e16 * 16, 16)] * 8
                for c in range(8):
                    v = plsc.load_gather(tb_v, [jv + c])
                    plsc.store_scatter(rows_b, [lane * 8 + (e16 * 128 + c)], v)
            pltpu.make_async_copy(
                ta_hbm.at[idx_v.at[pl.ds(off, CH)]], buf, sem).wait()
            pltpu.sync_copy(buf, outa_hbm.at[pl.ds(base + off, CH)])
            pltpu.sync_copy(rows_b, outb_hbm.at[pl.ds((base + off) * 8, CH * 8)])

        start(0, bufs[0], sems[0])

        def body(ih, carry):
            for bslot in range(2):
                ch = ih * 2 + bslot
                start(ch + 1, bufs[1 - bslot], sems[1 - bslot])
                finish(ch, bufs[bslot], sems[bslot])
            return carry

        lax.fori_loop(0, n_ch // 2, body, 0)
        finish(n_ch - 1, bufs[0], sems[0])

    return gather_kernel(table_a, table_b_flat, idx_flat)


def _unpack_pair(words):
    lo_f = lax.bitcast_convert_type(lax.shift_left(words, 16), jnp.float32)
    hi_f = lax.bitcast_convert_type(
        jnp.bitwise_and(words, jnp.int32(-65536)), jnp.float32)
    return lo_f, hi_f


def _compute_parts(atom_ref, aga_ref, agb_ref, nbr_ref, w0_ref, w2_ref, b_ref,
                   bn, m):
    sp = jnp.dot(atom_ref[:], w0_ref[:], preferred_element_type=jnp.float32)
    sp = sp + b_ref[:]
    e = jnp.dot(nbr_ref[:].reshape(m * bn, NBR), w2_ref[:],
                preferred_element_type=jnp.float32)
    qf, qc = _unpack_pair(aga_ref[:].reshape(m * bn, AFL))
    qn_lo, qn_hi = _unpack_pair(agb_ref[:].reshape(m * bn, 8))
    qn = jnp.concatenate([qn_lo, qn_hi], axis=1)

    def bcast(x):
        w = x.shape[1]
        return jnp.broadcast_to(x[None, :, :], (m, bn, w)).reshape(m * bn, w)

    f = bcast(sp[:, :AFL]) + qf + e[:, :AFL]
    c = bcast(sp[:, AFL:2 * AFL]) + qc + e[:, AFL:2 * AFL]
    nw = bcast(sp[:, 2 * AFL:]) + qn + e[:, 2 * AFL:]
    return f, c, nw


def _mm_stats_body(atom_ref, aga_ref, agb_ref, nbr_ref, w0_ref, w2_ref, b_ref,
                   s_ref, q_ref, *, bn, m):
    f, c, nw = _compute_parts(atom_ref, aga_ref, agb_ref, nbr_ref,
                              w0_ref, w2_ref, b_ref, bn, m)

    @pl.when(pl.program_id(0) == 0)
    def _():
        s_ref[:] = jnp.zeros_like(s_ref)
        q_ref[:] = jnp.zeros_like(q_ref)

    s_ref[:] += jnp.concatenate(
        [jnp.sum(f, axis=0, keepdims=True),
         jnp.sum(c, axis=0, keepdims=True),
         jnp.sum(nw, axis=0, keepdims=True)], axis=1)
    q_ref[:] += jnp.concatenate(
        [jnp.sum(f * f, axis=0, keepdims=True),
         jnp.sum(c * c, axis=0, keepdims=True),
         jnp.sum(nw * nw, axis=0, keepdims=True)], axis=1)


def _apply_body(atom_ref, aga_ref, agb_ref, nbr_ref, w0_ref, w2_ref, b_ref,
                s_ref, q_ref, g1_ref, b1_ref,
                ns_ref, nn_ref, s2_ref, q2_ref, *, bn, m, nm_total):
    f, c, nw = _compute_parts(atom_ref, aga_ref, agb_ref, nbr_ref,
                              w0_ref, w2_ref, b_ref, bn, m)

    mean = s_ref[:] / nm_total
    var = q_ref[:] / nm_total - mean * mean
    a1 = g1_ref[:] * lax.rsqrt(var + EPS)
    c1 = b1_ref[:] - mean * a1

    fb = (f * a1[:, :AFL] + c1[:, :AFL]).reshape(m, bn, AFL)
    cb = (c * a1[:, AFL:2 * AFL] + c1[:, AFL:2 * AFL]).reshape(m, bn, AFL)
    nwb = (nw * a1[:, 2 * AFL:] + c1[:, 2 * AFL:]).reshape(m, bn, NBR)

    mx = fb[0]
    for j in range(1, m):
        mx = jnp.maximum(mx, fb[j])
    z = jnp.zeros((bn, AFL), jnp.float32)
    acc = jnp.zeros((bn, AFL), jnp.float32)
    for j in range(m):
        e = jnp.exp(fb[j] - mx)
        z = z + e
        acc = acc + e * jnp.maximum(cb[j], 0.0)
    ns = acc / z
    ns_ref[:] = ns
    nn_ref[:] = nwb + nbr_ref[:]

    @pl.when(pl.program_id(0) == 0)
    def _():
        s2_ref[:] = jnp.zeros_like(s2_ref)
        q2_ref[:] = jnp.zeros_like(q2_ref)

    s2_ref[:] += jnp.sum(ns, axis=0, keepdims=True)
    q2_ref[:] += jnp.sum(ns * ns, axis=0, keepdims=True)


def _bn2_body(atom_ref, ns_ref, s2_ref, q2_ref, g2_ref, b2_ref, out_ref, *, n_total):
    mean = s2_ref[:] / n_total
    var = q2_ref[:] / n_total - mean * mean
    a2 = g2_ref[:] * lax.rsqrt(var + EPS)
    c2 = b2_ref[:] - mean * a2
    out_ref[:] = atom_ref[:] + ns_ref[:] * a2 + c2


def kernel(atom_in_fea, nbr_fea, nbr_fea_idx, W, b, g1, b1, g2, b2):
    N, M = nbr_fea_idx.shape
    E = N * M

    idx_flat = nbr_fea_idx.astype(jnp.int32).T.reshape(E)
    nbr_t = jnp.transpose(nbr_fea, (1, 0, 2))
    w0 = W[:AFL, :]
    w1 = W[AFL:2 * AFL, :]
    w2 = W[2 * AFL:, :]
    b2d = b.reshape(1, DF)

    BQ = 1000
    qta, qtb = pl.pallas_call(
        functools.partial(_qproj_body, bn=BQ),
        grid=(N // BQ,),
        in_specs=[
            pl.BlockSpec((BQ, AFL), lambda i: (i, 0)),
            pl.BlockSpec((AFL, DF), lambda i: (0, 0)),
        ],
        out_specs=[
            pl.BlockSpec((BQ, AFL), lambda i: (i, 0)),
            pl.BlockSpec((BQ, 8), lambda i: (i, 0)),
        ],
        out_shape=[
            jax.ShapeDtypeStruct((N, AFL), jnp.int32),
            jax.ShapeDtypeStruct((N, 8), jnp.int32),
        ],
    )(atom_in_fea, w1)

    aga, agb_flat = _sc_gather(qta, qtb.reshape(N * 8), idx_flat)
    aga3 = aga.reshape(M, N, AFL)
    agb3 = agb_flat.reshape(M, N, 8)

    BN = 200
    grid_a = N // BN

    dense_specs = [
        pl.BlockSpec((BN, AFL), lambda i: (i, 0)),
        pl.BlockSpec((M, BN, AFL), lambda i: (0, i, 0)),
        pl.BlockSpec((M, BN, 8), lambda i: (0, i, 0)),
        pl.BlockSpec((M, BN, NBR), lambda i: (0, i, 0)),
        pl.BlockSpec((AFL, DF), lambda i: (0, 0)),
        pl.BlockSpec((NBR, DF), lambda i: (0, 0)),
        pl.BlockSpec((1, DF), lambda i: (0, 0)),
    ]
    s, q = pl.pallas_call(
        functools.partial(_mm_stats_body, bn=BN, m=M),
        grid=(grid_a,),
        in_specs=dense_specs,
        out_specs=[
            pl.BlockSpec((1, DF), lambda i: (0, 0)),
            pl.BlockSpec((1, DF), lambda i: (0, 0)),
        ],
        out_shape=[
            jax.ShapeDtypeStruct((1, DF), jnp.float32),
            jax.ShapeDtypeStruct((1, DF), jnp.float32),
        ],
    )(atom_in_fea, aga3, agb3, nbr_t, w0, w2, b2d)

    ns, nn_t, s2, q2 = pl.pallas_call(
        functools.partial(_apply_body, bn=BN, m=M, nm_total=float(E)),
        grid=(grid_a,),
        in_specs=dense_specs + [
            pl.BlockSpec((1, DF), lambda i: (0, 0)),
            pl.BlockSpec((1, DF), lambda i: (0, 0)),
            pl.BlockSpec((1, DF), lambda i: (0, 0)),
            pl.BlockSpec((1, DF), lambda i: (0, 0)),
        ],
        out_specs=[
            pl.BlockSpec((BN, AFL), lambda i: (i, 0)),
            pl.BlockSpec((M, BN, NBR), lambda i: (0, i, 0)),
            pl.BlockSpec((1, AFL), lambda i: (0, 0)),
            pl.BlockSpec((1, AFL), lambda i: (0, 0)),
        ],
        out_shape=[
            jax.ShapeDtypeStruct((N, AFL), jnp.float32),
            jax.ShapeDtypeStruct((M, N, NBR), jnp.float32),
            jax.ShapeDtypeStruct((1, AFL), jnp.float32),
            jax.ShapeDtypeStruct((1, AFL), jnp.float32),
        ],
    )(atom_in_fea, aga3, agb3, nbr_t, w0, w2, b2d,
      s, q, g1.reshape(1, DF), b1.reshape(1, DF))
    nn = jnp.transpose(nn_t, (1, 0, 2))

    BC = 1000
    out = pl.pallas_call(
        functools.partial(_bn2_body, n_total=float(N)),
        grid=(N // BC,),
        in_specs=[
            pl.BlockSpec((BC, AFL), lambda i: (i, 0)),
            pl.BlockSpec((BC, AFL), lambda i: (i, 0)),
            pl.BlockSpec((1, AFL), lambda i: (0, 0)),
            pl.BlockSpec((1, AFL), lambda i: (0, 0)),
            pl.BlockSpec((1, AFL), lambda i: (0, 0)),
            pl.BlockSpec((1, AFL), lambda i: (0, 0)),
        ],
        out_specs=pl.BlockSpec((BC, AFL), lambda i: (i, 0)),
        out_shape=jax.ShapeDtypeStruct((N, AFL), jnp.float32),
    )(atom_in_fea, ns, s2, q2, g2.reshape(1, AFL), b2.reshape(1, AFL))

    return (out, nn)
